# Initial kernel scaffold; baseline (speedup 1.0000x reference)
import jax
import jax.numpy as jnp
from jax.experimental import pallas as pl


def kernel(x, edge_index, Wl1, Wr1, b1, g1, be1, Wl2, Wr2, b2, g2, be2, mW1, mb1, mg1, mbe1, mW2, mb2):
    raise NotImplementedError("write your pallas kernel here")


# speedup vs baseline: 5.0708x; 5.0708x over previous
import functools

import jax
import jax.numpy as jnp
from jax import lax
from jax.experimental import pallas as pl
from jax.experimental.pallas import tpu as pltpu
from jax.experimental.pallas import tpu_sc as plsc

N = 10000
E = 320000
D = 128
D_CAT = 3 * D
D_MLP_H = 256

NC = 2
NS = 16
NW = NC * NS
K = 128
CPW = 79
EPW = CPW * K
EP = NW * EPW
PADE = EP - E
NPAD = 10240
RPW = NPAD // NS
DEGW = 16

_HIGHEST = lax.Precision.HIGHEST


_SC_MESH = plsc.VectorSubcoreMesh(core_axis_name="c", subcore_axis_name="s")


def _sc_agg_body(feats, srcr, dstr, zrows, out,
                 src_v, dst_v, rows_v, acc_sh, sem):
    cid = lax.axis_index("c")
    sid = lax.axis_index("s")
    wid = sid * NC + cid

    pltpu.sync_copy(zrows, acc_sh.at[pl.ds(sid * RPW, RPW)])
    plsc.subcore_barrier()

    base = wid * EPW

    @pl.loop(0, CPW)
    def _(c):
        off = base + c * K
        pltpu.sync_copy(srcr.at[pl.ds(off, K)], src_v)
        pltpu.sync_copy(dstr.at[pl.ds(off, K)], dst_v)
        pltpu.async_copy(feats.at[src_v], rows_v, sem).wait()
        pltpu.sync_copy(rows_v, acc_sh.at[dst_v], add=True)

    plsc.subcore_barrier()
    pltpu.sync_copy(acc_sh.at[pl.ds(sid * RPW, RPW)],
                    out.at[cid, pl.ds(sid * RPW, RPW)])


_sc_agg = pl.kernel(
    _sc_agg_body, mesh=_SC_MESH,
    out_type=jax.ShapeDtypeStruct((NC, NPAD, D), jnp.float32),
    scratch_types=[
        pltpu.VMEM((K,), jnp.int32),
        pltpu.VMEM((K,), jnp.int32),
        pltpu.VMEM((K, D), jnp.float32),
        pltpu.VMEM_SHARED((NPAD, D), jnp.float32),
        pltpu.SemaphoreType.DMA,
    ])


def _sc_deg_body(dstr, zrows, ones, out, dst_v, ones_v, acc_sh):
    cid = lax.axis_index("c")
    sid = lax.axis_index("s")
    wid = sid * NC + cid

    pltpu.sync_copy(zrows, acc_sh.at[pl.ds(sid * RPW, RPW)])
    pltpu.sync_copy(ones, ones_v)
    plsc.subcore_barrier()

    base = wid * EPW

    @pl.loop(0, CPW)
    def _(c):
        off = base + c * K
        pltpu.sync_copy(dstr.at[pl.ds(off, K)], dst_v)
        pltpu.sync_copy(ones_v, acc_sh.at[dst_v], add=True)

    plsc.subcore_barrier()
    pltpu.sync_copy(acc_sh.at[pl.ds(sid * RPW, RPW)],
                    out.at[cid, pl.ds(sid * RPW, RPW)])


_sc_deg = pl.kernel(
    _sc_deg_body, mesh=_SC_MESH,
    out_type=jax.ShapeDtypeStruct((NC, NPAD, D), jnp.float32),
    scratch_types=[
        pltpu.VMEM((K,), jnp.int32),
        pltpu.VMEM((K, D), jnp.float32),
        pltpu.VMEM_SHARED((NPAD, D), jnp.float32),
    ])


BR = 1000
G = N // BR
_EPS = 1e-5


def _row_spec(width):
    return pl.BlockSpec((BR, width), lambda i: (i, 0))


def _full_spec(r, c):
    return pl.BlockSpec((r, c), lambda i: (0, 0))


def _bn_scale(stats):
    mu = stats[0:1, :] * (1.0 / N)
    var = stats[1:2, :] * (1.0 / N) - mu * mu
    return mu, lax.rsqrt(var + _EPS)


def _sage_pre_body(p0, p1, d0, d1, feats, Wl, Wr, b, hpre, stats, ssum, ssq):
    i = pl.program_id(0)
    agg = p0[...] + p1[...]
    deg = jnp.maximum(d0[...] + d1[...], 1.0)
    mean = agg / deg
    h = (jnp.dot(mean, Wl[...], precision=_HIGHEST)
         + jnp.dot(feats[...], Wr[...], precision=_HIGHEST)
         + b[...])
    hpre[...] = h

    @pl.when(i == 0)
    def _():
        ssum[...] = jnp.zeros_like(ssum)
        ssq[...] = jnp.zeros_like(ssq)

    ssum[...] += jnp.sum(h, axis=0, keepdims=True)
    ssq[...] += jnp.sum(h * h, axis=0, keepdims=True)

    @pl.when(i == G - 1)
    def _():
        stats[0:1, :] = ssum[...]
        stats[1:2, :] = ssq[...]


def _sage_pre(p0, p1, d0, d1, feats, Wl, Wr, b):
    return pl.pallas_call(
        _sage_pre_body,
        grid=(G,),
        in_specs=[_row_spec(D), _row_spec(D), _row_spec(1), _row_spec(1),
                  _row_spec(D), _full_spec(D, D), _full_spec(D, D),
                  _full_spec(1, D)],
        out_specs=[_row_spec(D), _full_spec(2, D)],
        out_shape=[jax.ShapeDtypeStruct((N, D), jnp.float32),
                   jax.ShapeDtypeStruct((2, D), jnp.float32)],
        scratch_shapes=[pltpu.VMEM((1, D), jnp.float32),
                        pltpu.VMEM((1, D), jnp.float32)],
    )(p0, p1, d0, d1, feats, Wl, Wr, b)


def _bn_relu_body(hpre, stats, g, be, o):
    mu, rstd = _bn_scale(stats)
    hn = (hpre[...] - mu) * rstd * g[...] + be[...]
    o[...] = jnp.maximum(hn, 0.0)


def _bn_relu(hpre, stats, g, be):
    return pl.pallas_call(
        _bn_relu_body,
        grid=(G,),
        in_specs=[_row_spec(D), _full_spec(2, D), _full_spec(1, D),
                  _full_spec(1, D)],
        out_specs=_row_spec(D),
        out_shape=jax.ShapeDtypeStruct((N, D), jnp.float32),
    )(hpre, stats, g, be)


def _bn2_head_pre_body(hpre2, stats2, g2, be2, xb, h1b, mW1, mb1,
                       z_out, zstats, zs, zq):
    i = pl.program_id(0)
    mu, rstd = _bn_scale(stats2)
    h2 = jnp.maximum((hpre2[...] - mu) * rstd * g2[...] + be2[...], 0.0)
    z = (jnp.dot(xb[...], mW1[0:D, :], precision=_HIGHEST)
         + jnp.dot(h1b[...], mW1[D:2 * D, :], precision=_HIGHEST)
         + jnp.dot(h2, mW1[2 * D:3 * D, :], precision=_HIGHEST)
         + mb1[...])
    z_out[...] = z

    @pl.when(i == 0)
    def _():
        zs[...] = jnp.zeros_like(zs)
        zq[...] = jnp.zeros_like(zq)

    zs[...] += jnp.sum(z, axis=0, keepdims=True)
    zq[...] += jnp.sum(z * z, axis=0, keepdims=True)

    @pl.when(i == G - 1)
    def _():
        zstats[0:1, :] = zs[...]
        zstats[1:2, :] = zq[...]


def _bn2_head_pre(hpre2, stats2, g2, be2, x, h1, mW1, mb1):
    return pl.pallas_call(
        _bn2_head_pre_body,
        grid=(G,),
        in_specs=[_row_spec(D), _full_spec(2, D), _full_spec(1, D),
                  _full_spec(1, D), _row_spec(D), _row_spec(D),
                  _full_spec(D_CAT, D_MLP_H), _full_spec(1, D_MLP_H)],
        out_specs=[_row_spec(D_MLP_H), _full_spec(2, D_MLP_H)],
        out_shape=[jax.ShapeDtypeStruct((N, D_MLP_H), jnp.float32),
                   jax.ShapeDtypeStruct((2, D_MLP_H), jnp.float32)],
        scratch_shapes=[pltpu.VMEM((1, D_MLP_H), jnp.float32),
                        pltpu.VMEM((1, D_MLP_H), jnp.float32)],
    )(hpre2, stats2, g2, be2, x, h1, mW1, mb1)


def _head_post_body(z, zstats, mg1, mbe1, mW2, mb2, o):
    mu, rstd = _bn_scale(zstats)
    zn = (z[...] - mu) * rstd * mg1[...] + mbe1[...]
    zl = jnp.where(zn > 0, zn, 0.01 * zn)
    o[...] = jnp.dot(zl, mW2[...], precision=_HIGHEST) + mb2[...]


def _head_post(z, zstats, mg1, mbe1, mW2, mb2):
    return pl.pallas_call(
        _head_post_body,
        grid=(G,),
        in_specs=[_row_spec(D_MLP_H), _full_spec(2, D_MLP_H),
                  _full_spec(1, D_MLP_H), _full_spec(1, D_MLP_H),
                  _full_spec(D_MLP_H, D), _full_spec(1, D)],
        out_specs=_row_spec(D),
        out_shape=jax.ShapeDtypeStruct((N, D), jnp.float32),
    )(z, zstats, mg1, mbe1, mW2, mb2)


def kernel(x, edge_index, Wl1, Wr1, b1, g1, be1, Wl2, Wr2, b2, g2, be2,
           mW1, mb1, mg1, mbe1, mW2, mb2):
    src = edge_index[0]
    dst = edge_index[1]
    pad = jnp.arange(PADE, dtype=jnp.int32)
    src_p = jnp.concatenate([src, pad % N])
    dst_p = jnp.concatenate([dst, N + pad % (NPAD - N)])
    zrows = jnp.zeros((RPW, D), jnp.float32)
    ones = jnp.ones((K, D), jnp.float32)

    degp = _sc_deg(dst_p, zrows, ones)
    aggp1 = _sc_agg(x, src_p, dst_p, zrows)
    d0 = degp[0, :N, 0:1]
    d1 = degp[1, :N, 0:1]
    hpre1, stats1 = _sage_pre(aggp1[0, :N], aggp1[1, :N], d0, d1, x,
                              Wl1, Wr1, b1.reshape(1, D))
    h1 = _bn_relu(hpre1, stats1, g1.reshape(1, D), be1.reshape(1, D))
    aggp2 = _sc_agg(h1, src_p, dst_p, zrows)
    hpre2, stats2 = _sage_pre(aggp2[0, :N], aggp2[1, :N], d0, d1, h1,
                              Wl2, Wr2, b2.reshape(1, D))
    z1, zstats = _bn2_head_pre(hpre2, stats2, g2.reshape(1, D),
                               be2.reshape(1, D), x, h1, mW1,
                               mb1.reshape(1, D_MLP_H))
    return _head_post(z1, zstats, mg1.reshape(1, D_MLP_H),
                      mbe1.reshape(1, D_MLP_H), mW2, mb2.reshape(1, D))


# speedup vs baseline: 6.4554x; 1.2731x over previous
import functools

import jax
import jax.numpy as jnp
from jax import lax
from jax.experimental import pallas as pl
from jax.experimental.pallas import tpu as pltpu
from jax.experimental.pallas import tpu_sc as plsc

N = 10000
E = 320000
D = 128
D_CAT = 3 * D
D_MLP_H = 256

NC = 2
NS = 16
NW = NC * NS
K = 128
CPW = 80
EPW = CPW * K
EP = NW * EPW
PADE = EP - E
MB = 1
NPAD = 10240
RPW = NPAD // NS
DEGW = 16

_HIGHEST = lax.Precision.HIGHEST


_SC_MESH = plsc.VectorSubcoreMesh(core_axis_name="c", subcore_axis_name="s")


def _copy_idx_row(idx_all, idx_v, c):
    for j in range(K // 16):
        sl = pl.ds(j * 16, 16)
        idx_v[sl] = idx_all[c, sl]


def _agg_phase(feats, src_all, dst_all, src_v, dst_v, rows_v, acc_sh,
               sem_g):
    @pl.loop(0, CPW)
    def _(c):
        _copy_idx_row(src_all, src_v, c)
        _copy_idx_row(dst_all, dst_v, c)
        pltpu.async_copy(feats.at[src_v], rows_v, sem_g).wait()
        pltpu.sync_copy(rows_v, acc_sh.at[dst_v], add=True)


def _sc_agg_body(feats, srcr, dstr, zrows, out,
                 src_all, dst_all, src_v, dst_v, rows_v, acc_sh, sem_g):
    cid = lax.axis_index("c")
    sid = lax.axis_index("s")
    wid = sid * NC + cid

    pltpu.sync_copy(zrows, acc_sh.at[pl.ds(sid * RPW, RPW)])
    pltpu.sync_copy(srcr.at[wid], src_all)
    pltpu.sync_copy(dstr.at[wid], dst_all)
    plsc.subcore_barrier()

    _agg_phase(feats, src_all, dst_all, src_v, dst_v, rows_v, acc_sh,
               sem_g)

    plsc.subcore_barrier()
    pltpu.sync_copy(acc_sh.at[pl.ds(sid * RPW, RPW)],
                    out.at[cid, pl.ds(sid * RPW, RPW)])


_sc_agg = pl.kernel(
    _sc_agg_body, mesh=_SC_MESH,
    out_type=jax.ShapeDtypeStruct((NC, NPAD, D), jnp.float32),
    scratch_types=[
        pltpu.VMEM((CPW, K), jnp.int32),
        pltpu.VMEM((CPW, K), jnp.int32),
        pltpu.VMEM((K,), jnp.int32),
        pltpu.VMEM((K,), jnp.int32),
        pltpu.VMEM((K, D), jnp.float32),
        pltpu.VMEM_SHARED((NPAD, D), jnp.float32),
        pltpu.SemaphoreType.DMA,
    ])


def _sc_agg_deg_body(feats, srcr, dstr, zrows, ones, out, degout,
                     src_all, dst_all, src_v, dst_v, rows_v, acc_sh,
                     sem_g):
    cid = lax.axis_index("c")
    sid = lax.axis_index("s")
    wid = sid * NC + cid
    myrows = pl.ds(sid * RPW, RPW)

    pltpu.sync_copy(zrows, acc_sh.at[myrows])
    pltpu.sync_copy(srcr.at[wid], src_all)
    pltpu.sync_copy(dstr.at[wid], dst_all)
    plsc.subcore_barrier()

    _agg_phase(feats, src_all, dst_all, src_v, dst_v, rows_v, acc_sh,
               sem_g)
    plsc.subcore_barrier()
    pltpu.sync_copy(acc_sh.at[myrows], out.at[cid, myrows])

    pltpu.sync_copy(zrows, acc_sh.at[myrows])
    pltpu.sync_copy(ones, rows_v)
    plsc.subcore_barrier()

    @pl.loop(0, CPW)
    def _(c):
        _copy_idx_row(dst_all, dst_v, c)
        pltpu.sync_copy(rows_v, acc_sh.at[dst_v], add=True)

    plsc.subcore_barrier()
    pltpu.sync_copy(acc_sh.at[myrows], degout.at[cid, myrows])


_sc_agg_deg = pl.kernel(
    _sc_agg_deg_body, mesh=_SC_MESH,
    out_type=[jax.ShapeDtypeStruct((NC, NPAD, D), jnp.float32),
              jax.ShapeDtypeStruct((NC, NPAD, D), jnp.float32)],
    scratch_types=[
        pltpu.VMEM((CPW, K), jnp.int32),
        pltpu.VMEM((CPW, K), jnp.int32),
        pltpu.VMEM((K,), jnp.int32),
        pltpu.VMEM((K,), jnp.int32),
        pltpu.VMEM((K, D), jnp.float32),
        pltpu.VMEM_SHARED((NPAD, D), jnp.float32),
        pltpu.SemaphoreType.DMA,
    ])


BR = 1000
G = N // BR
_EPS = 1e-5


def _row_spec(width):
    return pl.BlockSpec((BR, width), lambda i: (i, 0))


def _full_spec(r, c):
    return pl.BlockSpec((r, c), lambda i: (0, 0))


def _bn_scale(stats):
    mu = stats[0:1, :] * (1.0 / N)
    var = stats[1:2, :] * (1.0 / N) - mu * mu
    return mu, lax.rsqrt(var + _EPS)


def _sage_pre_body(p0, p1, d0, d1, feats, Wl, Wr, b, hpre, stats, ssum, ssq):
    i = pl.program_id(0)
    agg = p0[...] + p1[...]
    deg = jnp.maximum(d0[...] + d1[...], 1.0)
    mean = agg / deg
    h = (jnp.dot(mean, Wl[...], precision=_HIGHEST)
         + jnp.dot(feats[...], Wr[...], precision=_HIGHEST)
         + b[...])
    hpre[...] = h

    @pl.when(i == 0)
    def _():
        ssum[...] = jnp.zeros_like(ssum)
        ssq[...] = jnp.zeros_like(ssq)

    ssum[...] += jnp.sum(h, axis=0, keepdims=True)
    ssq[...] += jnp.sum(h * h, axis=0, keepdims=True)

    @pl.when(i == G - 1)
    def _():
        stats[0:1, :] = ssum[...]
        stats[1:2, :] = ssq[...]


def _sage_pre(p0, p1, d0, d1, feats, Wl, Wr, b):
    return pl.pallas_call(
        _sage_pre_body,
        grid=(G,),
        in_specs=[_row_spec(D), _row_spec(D), _row_spec(1), _row_spec(1),
                  _row_spec(D), _full_spec(D, D), _full_spec(D, D),
                  _full_spec(1, D)],
        out_specs=[_row_spec(D), _full_spec(2, D)],
        out_shape=[jax.ShapeDtypeStruct((N, D), jnp.float32),
                   jax.ShapeDtypeStruct((2, D), jnp.float32)],
        scratch_shapes=[pltpu.VMEM((1, D), jnp.float32),
                        pltpu.VMEM((1, D), jnp.float32)],
    )(p0, p1, d0, d1, feats, Wl, Wr, b)


def _bn_relu_body(hpre, stats, g, be, o):
    mu, rstd = _bn_scale(stats)
    hn = (hpre[...] - mu) * rstd * g[...] + be[...]
    o[...] = jnp.maximum(hn, 0.0)


def _bn_relu(hpre, stats, g, be):
    return pl.pallas_call(
        _bn_relu_body,
        grid=(G,),
        in_specs=[_row_spec(D), _full_spec(2, D), _full_spec(1, D),
                  _full_spec(1, D)],
        out_specs=_row_spec(D),
        out_shape=jax.ShapeDtypeStruct((N, D), jnp.float32),
    )(hpre, stats, g, be)


def _bn2_head_pre_body(hpre2, stats2, g2, be2, xb, h1b, mW1, mb1,
                       z_out, zstats, zs, zq):
    i = pl.program_id(0)
    mu, rstd = _bn_scale(stats2)
    h2 = jnp.maximum((hpre2[...] - mu) * rstd * g2[...] + be2[...], 0.0)
    z = (jnp.dot(xb[...], mW1[0:D, :], precision=_HIGHEST)
         + jnp.dot(h1b[...], mW1[D:2 * D, :], precision=_HIGHEST)
         + jnp.dot(h2, mW1[2 * D:3 * D, :], precision=_HIGHEST)
         + mb1[...])
    z_out[...] = z

    @pl.when(i == 0)
    def _():
        zs[...] = jnp.zeros_like(zs)
        zq[...] = jnp.zeros_like(zq)

    zs[...] += jnp.sum(z, axis=0, keepdims=True)
    zq[...] += jnp.sum(z * z, axis=0, keepdims=True)

    @pl.when(i == G - 1)
    def _():
        zstats[0:1, :] = zs[...]
        zstats[1:2, :] = zq[...]


def _bn2_head_pre(hpre2, stats2, g2, be2, x, h1, mW1, mb1):
    return pl.pallas_call(
        _bn2_head_pre_body,
        grid=(G,),
        in_specs=[_row_spec(D), _full_spec(2, D), _full_spec(1, D),
                  _full_spec(1, D), _row_spec(D), _row_spec(D),
                  _full_spec(D_CAT, D_MLP_H), _full_spec(1, D_MLP_H)],
        out_specs=[_row_spec(D_MLP_H), _full_spec(2, D_MLP_H)],
        out_shape=[jax.ShapeDtypeStruct((N, D_MLP_H), jnp.float32),
                   jax.ShapeDtypeStruct((2, D_MLP_H), jnp.float32)],
        scratch_shapes=[pltpu.VMEM((1, D_MLP_H), jnp.float32),
                        pltpu.VMEM((1, D_MLP_H), jnp.float32)],
    )(hpre2, stats2, g2, be2, x, h1, mW1, mb1)


def _head_post_body(z, zstats, mg1, mbe1, mW2, mb2, o):
    mu, rstd = _bn_scale(zstats)
    zn = (z[...] - mu) * rstd * mg1[...] + mbe1[...]
    zl = jnp.where(zn > 0, zn, 0.01 * zn)
    o[...] = jnp.dot(zl, mW2[...], precision=_HIGHEST) + mb2[...]


def _head_post(z, zstats, mg1, mbe1, mW2, mb2):
    return pl.pallas_call(
        _head_post_body,
        grid=(G,),
        in_specs=[_row_spec(D_MLP_H), _full_spec(2, D_MLP_H),
                  _full_spec(1, D_MLP_H), _full_spec(1, D_MLP_H),
                  _full_spec(D_MLP_H, D), _full_spec(1, D)],
        out_specs=_row_spec(D),
        out_shape=jax.ShapeDtypeStruct((N, D), jnp.float32),
    )(z, zstats, mg1, mbe1, mW2, mb2)


def kernel(x, edge_index, Wl1, Wr1, b1, g1, be1, Wl2, Wr2, b2, g2, be2,
           mW1, mb1, mg1, mbe1, mW2, mb2):
    src = edge_index[0]
    dst = edge_index[1]
    pad = jnp.arange(PADE, dtype=jnp.int32)
    src_p = jnp.concatenate([src, pad % N]).reshape(NW, CPW, K)
    dst_p = jnp.concatenate([dst, N + pad % (NPAD - N)]).reshape(NW, CPW, K)
    zrows = jnp.zeros((RPW, D), jnp.float32)
    ones = jnp.ones((K, D), jnp.float32)

    aggp1, degp = _sc_agg_deg(x, src_p, dst_p, zrows, ones)
    d0 = degp[0, :N, 0:1]
    d1 = degp[1, :N, 0:1]
    hpre1, stats1 = _sage_pre(aggp1[0, :N], aggp1[1, :N], d0, d1, x,
                              Wl1, Wr1, b1.reshape(1, D))
    h1 = _bn_relu(hpre1, stats1, g1.reshape(1, D), be1.reshape(1, D))
    aggp2 = _sc_agg(h1, src_p, dst_p, zrows)
    hpre2, stats2 = _sage_pre(aggp2[0, :N], aggp2[1, :N], d0, d1, h1,
                              Wl2, Wr2, b2.reshape(1, D))
    z1, zstats = _bn2_head_pre(hpre2, stats2, g2.reshape(1, D),
                               be2.reshape(1, D), x, h1, mW1,
                               mb1.reshape(1, D_MLP_H))
    return _head_post(z1, zstats, mg1.reshape(1, D_MLP_H),
                      mbe1.reshape(1, D_MLP_H), mW2, mb2.reshape(1, D))


# speedup vs baseline: 6.8936x; 1.0679x over previous
import functools

import jax
import jax.numpy as jnp
from jax import lax
from jax.experimental import pallas as pl
from jax.experimental.pallas import tpu as pltpu
from jax.experimental.pallas import tpu_sc as plsc

N = 10000
E = 320000
D = 128
D_CAT = 3 * D
D_MLP_H = 256

NC = 2
NS = 16
NW = NC * NS
K = 128
CPW = 80
EPW = CPW * K
EP = NW * EPW
PADE = EP - E
MB = 1
NPAD = 10240
RPW = NPAD // NS
DEGW = 16

_HIGHEST = lax.Precision.HIGHEST


_SC_MESH = plsc.VectorSubcoreMesh(core_axis_name="c", subcore_axis_name="s")


KH = K // 2


def _copy_idx_half(idx_all, idx_v, c, off):
    for j in range(KH // 16):
        idx_v[pl.ds(j * 16, 16)] = idx_all[c, pl.ds(off + j * 16, 16)]


def _agg_phase(feats, src_all, dst_all, src_va, src_vb, dst_v, rows_v,
               acc_sh, sem_g):
    rows_a = rows_v.at[pl.ds(0, KH)]
    rows_b = rows_v.at[pl.ds(KH, KH)]

    @pl.loop(0, CPW)
    def _(c):
        _copy_idx_half(src_all, src_va, c, 0)
        ga = pltpu.async_copy(feats.at[src_va], rows_a, sem_g)
        _copy_idx_half(src_all, src_vb, c, KH)
        gb = pltpu.async_copy(feats.at[src_vb], rows_b, sem_g)
        _copy_idx_half(dst_all, dst_v, c, 0)
        ga.wait()
        pltpu.sync_copy(rows_a, acc_sh.at[dst_v], add=True)
        _copy_idx_half(dst_all, dst_v, c, KH)
        gb.wait()
        pltpu.sync_copy(rows_b, acc_sh.at[dst_v], add=True)


def _sc_agg_body(feats, srcr, dstr, zrows, out,
                 src_all, dst_all, src_va, src_vb, dst_v, rows_v, acc_sh,
                 sem_g):
    cid = lax.axis_index("c")
    sid = lax.axis_index("s")
    wid = sid * NC + cid

    pltpu.sync_copy(zrows, acc_sh.at[pl.ds(sid * RPW, RPW)])
    pltpu.sync_copy(srcr.at[wid], src_all)
    pltpu.sync_copy(dstr.at[wid], dst_all)
    plsc.subcore_barrier()

    _agg_phase(feats, src_all, dst_all, src_va, src_vb, dst_v, rows_v,
               acc_sh, sem_g)

    plsc.subcore_barrier()
    pltpu.sync_copy(acc_sh.at[pl.ds(sid * RPW, RPW)],
                    out.at[cid, pl.ds(sid * RPW, RPW)])


_sc_agg = pl.kernel(
    _sc_agg_body, mesh=_SC_MESH,
    out_type=jax.ShapeDtypeStruct((NC, NPAD, D), jnp.float32),
    scratch_types=[
        pltpu.VMEM((CPW, K), jnp.int32),
        pltpu.VMEM((CPW, K), jnp.int32),
        pltpu.VMEM((KH,), jnp.int32),
        pltpu.VMEM((KH,), jnp.int32),
        pltpu.VMEM((KH,), jnp.int32),
        pltpu.VMEM((K, D), jnp.float32),
        pltpu.VMEM_SHARED((NPAD, D), jnp.float32),
        pltpu.SemaphoreType.DMA,
    ])


def _sc_agg_deg_body(feats, srcr, dstr, zrows, ones, out, degout,
                     src_all, dst_all, src_va, src_vb, dst_v, rows_v,
                     acc_sh, sem_g):
    cid = lax.axis_index("c")
    sid = lax.axis_index("s")
    wid = sid * NC + cid
    myrows = pl.ds(sid * RPW, RPW)

    pltpu.sync_copy(zrows, acc_sh.at[myrows])
    pltpu.sync_copy(srcr.at[wid], src_all)
    pltpu.sync_copy(dstr.at[wid], dst_all)
    plsc.subcore_barrier()

    _agg_phase(feats, src_all, dst_all, src_va, src_vb, dst_v, rows_v,
               acc_sh, sem_g)
    plsc.subcore_barrier()
    pltpu.sync_copy(acc_sh.at[myrows], out.at[cid, myrows])

    pltpu.sync_copy(zrows, acc_sh.at[myrows])
    pltpu.sync_copy(ones, rows_v.at[pl.ds(0, KH)])
    plsc.subcore_barrier()

    @pl.loop(0, CPW)
    def _(c):
        _copy_idx_half(dst_all, dst_v, c, 0)
        pltpu.sync_copy(rows_v.at[pl.ds(0, KH)], acc_sh.at[dst_v], add=True)
        _copy_idx_half(dst_all, dst_v, c, KH)
        pltpu.sync_copy(rows_v.at[pl.ds(0, KH)], acc_sh.at[dst_v], add=True)

    plsc.subcore_barrier()
    pltpu.sync_copy(acc_sh.at[myrows], degout.at[cid, myrows])


_sc_agg_deg = pl.kernel(
    _sc_agg_deg_body, mesh=_SC_MESH,
    out_type=[jax.ShapeDtypeStruct((NC, NPAD, D), jnp.float32),
              jax.ShapeDtypeStruct((NC, NPAD, D), jnp.float32)],
    scratch_types=[
        pltpu.VMEM((CPW, K), jnp.int32),
        pltpu.VMEM((CPW, K), jnp.int32),
        pltpu.VMEM((KH,), jnp.int32),
        pltpu.VMEM((KH,), jnp.int32),
        pltpu.VMEM((KH,), jnp.int32),
        pltpu.VMEM((K, D), jnp.float32),
        pltpu.VMEM_SHARED((NPAD, D), jnp.float32),
        pltpu.SemaphoreType.DMA,
    ])


BR = 1000
G = N // BR
_EPS = 1e-5


def _row_spec(width):
    return pl.BlockSpec((BR, width), lambda i: (i, 0))


def _full_spec(r, c):
    return pl.BlockSpec((r, c), lambda i: (0, 0))


def _bn_scale(stats):
    mu = stats[0:1, :] * (1.0 / N)
    var = stats[1:2, :] * (1.0 / N) - mu * mu
    return mu, lax.rsqrt(var + _EPS)


def _sage_pre_body(p0, p1, d0, d1, feats, Wl, Wr, b, hpre, stats, ssum, ssq):
    i = pl.program_id(0)
    agg = p0[...] + p1[...]
    deg = jnp.maximum(d0[...] + d1[...], 1.0)
    mean = agg / deg
    h = (jnp.dot(mean, Wl[...], precision=_HIGHEST)
         + jnp.dot(feats[...], Wr[...], precision=_HIGHEST)
         + b[...])
    hpre[...] = h

    @pl.when(i == 0)
    def _():
        ssum[...] = jnp.zeros_like(ssum)
        ssq[...] = jnp.zeros_like(ssq)

    ssum[...] += jnp.sum(h, axis=0, keepdims=True)
    ssq[...] += jnp.sum(h * h, axis=0, keepdims=True)

    @pl.when(i == G - 1)
    def _():
        stats[0:1, :] = ssum[...]
        stats[1:2, :] = ssq[...]


def _sage_pre(p0, p1, d0, d1, feats, Wl, Wr, b):
    return pl.pallas_call(
        _sage_pre_body,
        grid=(G,),
        in_specs=[_row_spec(D), _row_spec(D), _row_spec(1), _row_spec(1),
                  _row_spec(D), _full_spec(D, D), _full_spec(D, D),
                  _full_spec(1, D)],
        out_specs=[_row_spec(D), _full_spec(2, D)],
        out_shape=[jax.ShapeDtypeStruct((N, D), jnp.float32),
                   jax.ShapeDtypeStruct((2, D), jnp.float32)],
        scratch_shapes=[pltpu.VMEM((1, D), jnp.float32),
                        pltpu.VMEM((1, D), jnp.float32)],
    )(p0, p1, d0, d1, feats, Wl, Wr, b)


def _bn_relu_body(hpre, stats, g, be, o):
    mu, rstd = _bn_scale(stats)
    hn = (hpre[...] - mu) * rstd * g[...] + be[...]
    o[...] = jnp.maximum(hn, 0.0)


def _bn_relu(hpre, stats, g, be):
    return pl.pallas_call(
        _bn_relu_body,
        grid=(G,),
        in_specs=[_row_spec(D), _full_spec(2, D), _full_spec(1, D),
                  _full_spec(1, D)],
        out_specs=_row_spec(D),
        out_shape=jax.ShapeDtypeStruct((N, D), jnp.float32),
    )(hpre, stats, g, be)


def _bn2_head_pre_body(hpre2, stats2, g2, be2, xb, h1b, mW1, mb1,
                       z_out, zstats, zs, zq):
    i = pl.program_id(0)
    mu, rstd = _bn_scale(stats2)
    h2 = jnp.maximum((hpre2[...] - mu) * rstd * g2[...] + be2[...], 0.0)
    z = (jnp.dot(xb[...], mW1[0:D, :], precision=_HIGHEST)
         + jnp.dot(h1b[...], mW1[D:2 * D, :], precision=_HIGHEST)
         + jnp.dot(h2, mW1[2 * D:3 * D, :], precision=_HIGHEST)
         + mb1[...])
    z_out[...] = z

    @pl.when(i == 0)
    def _():
        zs[...] = jnp.zeros_like(zs)
        zq[...] = jnp.zeros_like(zq)

    zs[...] += jnp.sum(z, axis=0, keepdims=True)
    zq[...] += jnp.sum(z * z, axis=0, keepdims=True)

    @pl.when(i == G - 1)
    def _():
        zstats[0:1, :] = zs[...]
        zstats[1:2, :] = zq[...]


def _bn2_head_pre(hpre2, stats2, g2, be2, x, h1, mW1, mb1):
    return pl.pallas_call(
        _bn2_head_pre_body,
        grid=(G,),
        in_specs=[_row_spec(D), _full_spec(2, D), _full_spec(1, D),
                  _full_spec(1, D), _row_spec(D), _row_spec(D),
                  _full_spec(D_CAT, D_MLP_H), _full_spec(1, D_MLP_H)],
        out_specs=[_row_spec(D_MLP_H), _full_spec(2, D_MLP_H)],
        out_shape=[jax.ShapeDtypeStruct((N, D_MLP_H), jnp.float32),
                   jax.ShapeDtypeStruct((2, D_MLP_H), jnp.float32)],
        scratch_shapes=[pltpu.VMEM((1, D_MLP_H), jnp.float32),
                        pltpu.VMEM((1, D_MLP_H), jnp.float32)],
    )(hpre2, stats2, g2, be2, x, h1, mW1, mb1)


def _head_post_body(z, zstats, mg1, mbe1, mW2, mb2, o):
    mu, rstd = _bn_scale(zstats)
    zn = (z[...] - mu) * rstd * mg1[...] + mbe1[...]
    zl = jnp.where(zn > 0, zn, 0.01 * zn)
    o[...] = jnp.dot(zl, mW2[...], precision=_HIGHEST) + mb2[...]


def _head_post(z, zstats, mg1, mbe1, mW2, mb2):
    return pl.pallas_call(
        _head_post_body,
        grid=(G,),
        in_specs=[_row_spec(D_MLP_H), _full_spec(2, D_MLP_H),
                  _full_spec(1, D_MLP_H), _full_spec(1, D_MLP_H),
                  _full_spec(D_MLP_H, D), _full_spec(1, D)],
        out_specs=_row_spec(D),
        out_shape=jax.ShapeDtypeStruct((N, D), jnp.float32),
    )(z, zstats, mg1, mbe1, mW2, mb2)


def kernel(x, edge_index, Wl1, Wr1, b1, g1, be1, Wl2, Wr2, b2, g2, be2,
           mW1, mb1, mg1, mbe1, mW2, mb2):
    src = edge_index[0]
    dst = edge_index[1]
    pad = jnp.arange(PADE, dtype=jnp.int32)
    src_p = jnp.concatenate([src, pad % N]).reshape(NW, CPW, K)
    dst_p = jnp.concatenate([dst, N + pad % (NPAD - N)]).reshape(NW, CPW, K)
    zrows = jnp.zeros((RPW, D), jnp.float32)
    ones = jnp.ones((K // 2, D), jnp.float32)

    aggp1, degp = _sc_agg_deg(x, src_p, dst_p, zrows, ones)
    d0 = degp[0, :N, 0:1]
    d1 = degp[1, :N, 0:1]
    hpre1, stats1 = _sage_pre(aggp1[0, :N], aggp1[1, :N], d0, d1, x,
                              Wl1, Wr1, b1.reshape(1, D))
    h1 = _bn_relu(hpre1, stats1, g1.reshape(1, D), be1.reshape(1, D))
    aggp2 = _sc_agg(h1, src_p, dst_p, zrows)
    hpre2, stats2 = _sage_pre(aggp2[0, :N], aggp2[1, :N], d0, d1, h1,
                              Wl2, Wr2, b2.reshape(1, D))
    z1, zstats = _bn2_head_pre(hpre2, stats2, g2.reshape(1, D),
                               be2.reshape(1, D), x, h1, mW1,
                               mb1.reshape(1, D_MLP_H))
    return _head_post(z1, zstats, mg1.reshape(1, D_MLP_H),
                      mbe1.reshape(1, D_MLP_H), mW2, mb2.reshape(1, D))
